# Initial kernel scaffold; baseline (speedup 1.0000x reference)
import jax
import jax.numpy as jnp
from jax.experimental import pallas as pl


def kernel(z_e, codebook):
    raise NotImplementedError("write your pallas kernel here")


# speedup vs baseline: 1.0077x; 1.0077x over previous
import functools

import jax
import jax.numpy as jnp
from jax import lax
from jax.experimental import pallas as pl
from jax.experimental.pallas import tpu as pltpu
from jax.experimental.pallas import tpu_sc as plsc

_B, _D, _HW = 16, 64, 1024
_K = 1024
_NC, _NS = 2, 16


def _vq_tc_body(z_ref, cb_ref, codes_ref, loss_ref):
    b = pl.program_id(0)
    z = z_ref[0]
    c = cb_ref[...]
    s = jnp.dot(c.astype(jnp.bfloat16), z.astype(jnp.bfloat16),
                preferred_element_type=jnp.float32)
    cn = jnp.sum(c * c, axis=1, keepdims=True)
    dist = cn - 2.0 * s
    m = jnp.min(dist, axis=0)
    row = lax.broadcasted_iota(jnp.int32, (_K, _HW), 0)
    codes = jnp.min(jnp.where(dist == m[None, :], row, jnp.int32(2**30)),
                    axis=0)
    codes_ref[0, 0, :] = codes
    part = jnp.reshape(jnp.sum(m) + jnp.sum(z * z), (1, 1))

    @pl.when(b == 0)
    def _():
        loss_ref[...] = part

    @pl.when(b > 0)
    def _():
        loss_ref[...] += part


def _vq_sc_gather(cb_hbm, codes_hbm, out_hbm, cb_v, codes_v, out_v):
    b = lax.axis_index("s")
    half = lax.axis_index("c")
    base_d = half * (_D // 2)
    pltpu.sync_copy(cb_hbm, cb_v)
    pltpu.sync_copy(codes_hbm.at[pl.ds(b * _HW, _HW)], codes_v)

    def body(j, carry):
        code16 = codes_v[pl.ds(j * 16, 16)] * _D + base_d
        for dd in range(_D // 2):
            out_v[dd, pl.ds(j * 16, 16)] = plsc.load_gather(
                cb_v, [code16 + dd])
        return carry

    lax.fori_loop(0, _HW // 16, body, 0)
    pltpu.sync_copy(out_v, out_hbm.at[pl.ds(b * _D + base_d, _D // 2)])


def _sc_gather_call(codebook, codes_flat):
    mesh = plsc.VectorSubcoreMesh(core_axis_name="c", subcore_axis_name="s")
    fn = functools.partial(
        pl.kernel,
        mesh=mesh,
        out_type=jax.ShapeDtypeStruct((_B * _D, _HW), jnp.float32),
        scratch_types=[
            pltpu.VMEM((_K * _D,), jnp.float32),
            pltpu.VMEM((_HW,), jnp.int32),
            pltpu.VMEM((_D // 2, _HW), jnp.float32),
        ],
        compiler_params=pltpu.CompilerParams(needs_layout_passes=False),
    )(_vq_sc_gather)
    return fn(codebook.reshape(_K * _D), codes_flat)


def kernel(z_e, codebook):
    B, D, H, W = z_e.shape
    z3 = z_e.reshape(B, D, H * W)
    codes3, loss_arr = pl.pallas_call(
        _vq_tc_body,
        grid=(B,),
        in_specs=[
            pl.BlockSpec((1, D, H * W), lambda b: (b, 0, 0)),
            pl.BlockSpec((_K, D), lambda b: (0, 0)),
        ],
        out_specs=[
            pl.BlockSpec((1, 1, H * W), lambda b: (b, 0, 0)),
            pl.BlockSpec((1, 1), lambda b: (0, 0)),
        ],
        out_shape=[
            jax.ShapeDtypeStruct((B, 1, H * W), jnp.int32),
            jax.ShapeDtypeStruct((1, 1), jnp.float32),
        ],
        compiler_params=pltpu.CompilerParams(
            dimension_semantics=("arbitrary",)),
    )(z3, codebook)

    codes_flat = codes3.reshape(B * H * W)
    zq = _sc_gather_call(codebook, codes_flat)
    z_q = zq.reshape(B, D, H, W)
    indices = codes3.reshape(B, H, W)
    vq_loss = (1.25 / (B * D * H * W)) * loss_arr[0, 0]
    return (z_q, indices, vq_loss)


# speedup vs baseline: 1.1315x; 1.1229x over previous
import functools

import jax
import jax.numpy as jnp
from jax import lax
from jax.experimental import pallas as pl
from jax.experimental.pallas import tpu as pltpu
from jax.experimental.pallas import tpu_sc as plsc

_B, _D, _HW = 16, 64, 1024
_K = 1024
_NC, _NS = 2, 16


def _vq_tc_body(z_ref, cb_ref, codes_ref, loss_ref):
    b = pl.program_id(0)
    z = z_ref[0]
    c = cb_ref[...]
    s = jnp.dot(c.astype(jnp.bfloat16), z.astype(jnp.bfloat16),
                preferred_element_type=jnp.float32)
    cn = jnp.sum(c * c, axis=1, keepdims=True)
    dist = cn - 2.0 * s
    m = jnp.min(dist, axis=0)
    row = lax.broadcasted_iota(jnp.int32, (_K, _HW), 0)
    codes = jnp.min(jnp.where(dist == m[None, :], row, jnp.int32(2**30)),
                    axis=0)
    codes_ref[0, 0, :] = codes
    part = jnp.reshape(jnp.sum(m) + jnp.sum(z * z), (1, 1))

    @pl.when(b == 0)
    def _():
        loss_ref[...] = part

    @pl.when(b > 0)
    def _():
        loss_ref[...] += part


def _vq_sc_gather(cb_hbm, codes_hbm, out_hbm, cb_v, codes_v, out_v):
    b = lax.axis_index("s")
    half = lax.axis_index("c")
    base_d = half * (_D // 2)
    pltpu.sync_copy(cb_hbm, cb_v)
    pltpu.sync_copy(codes_hbm.at[pl.ds(b * _HW, _HW)], codes_v)

    @plsc.parallel_loop(0, _HW // 16, unroll=2)
    def _(j):
        code16 = codes_v[pl.ds(j * 16, 16)] * _D + base_d
        for dd in range(_D // 2):
            out_v[dd, pl.ds(j * 16, 16)] = plsc.load_gather(
                cb_v, [code16 + dd])
    pltpu.sync_copy(out_v, out_hbm.at[pl.ds(b * _D + base_d, _D // 2)])


def _sc_gather_call(codebook, codes_flat):
    mesh = plsc.VectorSubcoreMesh(core_axis_name="c", subcore_axis_name="s")
    fn = functools.partial(
        pl.kernel,
        mesh=mesh,
        out_type=jax.ShapeDtypeStruct((_B * _D, _HW), jnp.float32),
        scratch_types=[
            pltpu.VMEM((_K * _D,), jnp.float32),
            pltpu.VMEM((_HW,), jnp.int32),
            pltpu.VMEM((_D // 2, _HW), jnp.float32),
        ],
        compiler_params=pltpu.CompilerParams(needs_layout_passes=False),
    )(_vq_sc_gather)
    return fn(codebook.reshape(_K * _D), codes_flat)


def kernel(z_e, codebook):
    B, D, H, W = z_e.shape
    z3 = z_e.reshape(B, D, H * W)
    codes3, loss_arr = pl.pallas_call(
        _vq_tc_body,
        grid=(B,),
        in_specs=[
            pl.BlockSpec((1, D, H * W), lambda b: (b, 0, 0)),
            pl.BlockSpec((_K, D), lambda b: (0, 0)),
        ],
        out_specs=[
            pl.BlockSpec((1, 1, H * W), lambda b: (b, 0, 0)),
            pl.BlockSpec((1, 1), lambda b: (0, 0)),
        ],
        out_shape=[
            jax.ShapeDtypeStruct((B, 1, H * W), jnp.int32),
            jax.ShapeDtypeStruct((1, 1), jnp.float32),
        ],
        compiler_params=pltpu.CompilerParams(
            dimension_semantics=("arbitrary",)),
    )(z3, codebook)

    codes_flat = codes3.reshape(B * H * W)
    zq = _sc_gather_call(codebook, codes_flat)
    z_q = zq.reshape(B, D, H, W)
    indices = codes3.reshape(B, H, W)
    vq_loss = (1.25 / (B * D * H * W)) * loss_arr[0, 0]
    return (z_q, indices, vq_loss)


# speedup vs baseline: 1.2707x; 1.1230x over previous
import functools

import jax
import jax.numpy as jnp
from jax import lax
from jax.experimental import pallas as pl
from jax.experimental.pallas import tpu as pltpu
from jax.experimental.pallas import tpu_sc as plsc

_B, _D, _HW = 16, 64, 1024
_K = 1024
_NC, _NS = 2, 16


def _vq_tc_body(z_ref, cb_ref, codes_ref, loss_ref):
    b = pl.program_id(0)
    z = z_ref[0]
    c = cb_ref[...]
    s = jnp.dot(c.astype(jnp.bfloat16), z.astype(jnp.bfloat16),
                preferred_element_type=jnp.float32)
    cn = jnp.sum(c * c, axis=1, keepdims=True)
    dist = cn - 2.0 * s
    m = jnp.min(dist, axis=0)
    row = lax.broadcasted_iota(jnp.int32, (_K, _HW), 0)
    codes = jnp.min(jnp.where(dist == m[None, :], row, jnp.int32(2**30)),
                    axis=0)
    codes_ref[0, 0, :] = codes
    part = jnp.reshape(jnp.sum(m) + jnp.sum(z * z), (1, 1))

    @pl.when(b == 0)
    def _():
        loss_ref[...] = part

    @pl.when(b > 0)
    def _():
        loss_ref[...] += part


def _vq_sc_gather(cb_hbm, codes_hbm, out_hbm, cb_v, codes_v, out_v):
    b = lax.axis_index("s")
    half = lax.axis_index("c")
    base_d = half * (_D // 2)
    pltpu.sync_copy(cb_hbm, cb_v)
    pltpu.sync_copy(codes_hbm.at[pl.ds(b * _HW, _HW)], codes_v)

    @plsc.parallel_loop(0, _HW // 16, unroll=2)
    def _(j):
        code16 = codes_v[pl.ds(j * 16, 16)] * _D + base_d
        for dd in range(_D // 2):
            out_v[dd, pl.ds(j * 16, 16)] = plsc.load_gather(
                cb_v, [code16 + dd])
    pltpu.sync_copy(out_v, out_hbm.at[b, pl.ds(base_d, _D // 2)])


def _sc_gather_call(codebook, codes_flat):
    mesh = plsc.VectorSubcoreMesh(core_axis_name="c", subcore_axis_name="s")
    fn = functools.partial(
        pl.kernel,
        mesh=mesh,
        out_type=jax.ShapeDtypeStruct((_B, _D, _HW), jnp.float32),
        scratch_types=[
            pltpu.VMEM((_K * _D,), jnp.float32),
            pltpu.VMEM((_HW,), jnp.int32),
            pltpu.VMEM((_D // 2, _HW), jnp.float32),
        ],
        compiler_params=pltpu.CompilerParams(needs_layout_passes=False),
    )(_vq_sc_gather)
    return fn(codebook.reshape(_K * _D), codes_flat)


def kernel(z_e, codebook):
    B, D, H, W = z_e.shape
    z3 = z_e.reshape(B, D, H * W)
    codes3, loss_arr = pl.pallas_call(
        _vq_tc_body,
        grid=(B,),
        in_specs=[
            pl.BlockSpec((1, D, H * W), lambda b: (b, 0, 0)),
            pl.BlockSpec((_K, D), lambda b: (0, 0)),
        ],
        out_specs=[
            pl.BlockSpec((1, 1, H * W), lambda b: (b, 0, 0)),
            pl.BlockSpec((1, 1), lambda b: (0, 0)),
        ],
        out_shape=[
            jax.ShapeDtypeStruct((B, 1, H * W), jnp.int32),
            jax.ShapeDtypeStruct((1, 1), jnp.float32),
        ],
        compiler_params=pltpu.CompilerParams(
            dimension_semantics=("arbitrary",)),
    )(z3, codebook)

    codes_flat = codes3.reshape(B * H * W)
    zq3 = _sc_gather_call(codebook, codes_flat)
    z_q = zq3.reshape(B, D, H, W)
    indices = codes3.reshape(B, H, W)
    vq_loss = (1.25 / (B * D * H * W)) * loss_arr[0, 0]
    return (z_q, indices, vq_loss)
